# Initial kernel scaffold; baseline (speedup 1.0000x reference)
import jax
import jax.numpy as jnp
from jax.experimental import pallas as pl


def kernel(V, E, weight, Wc, bc, SW, Sb, gamma_V, beta_V, gamma_E, beta_E, edge_index):
    raise NotImplementedError("write your pallas kernel here")


# speedup vs baseline: 1.0002x; 1.0002x over previous
import jax
import jax.numpy as jnp
from jax.experimental import pallas as pl
from jax.experimental.pallas import tpu as pltpu

LEAKY = 0.1
EPS = 1e-5
N = 10000
EDGES = 320000
D_V = 128
D_E = 16
CELL_ARCH = [(0, 1, 0), (0, 2, 1), (1, 2, 2), (0, 3, 3), (2, 3, 4), (0, 4, 5), (3, 4, 6)]

ROW_BLK = 1000


def _final_v_kernel(vcat_ref, wc_ref, bc_ref, gv_ref, bv_ref, v_ref, sws_ref, swd_ref,
                    vout_ref, p_ref, q_ref):
    vnew = jnp.dot(vcat_ref[...], wc_ref[...], preferred_element_type=jnp.float32) + bc_ref[...]
    inv = 1.0 / jnp.sqrt(1.0 + EPS)
    vbn = vnew * inv * gv_ref[...] + bv_ref[...]
    vact = jnp.where(vbn >= 0, vbn, LEAKY * vbn)
    vout_ref[...] = vact + v_ref[...]
    p_ref[...] = jnp.dot(vnew, sws_ref[...], preferred_element_type=jnp.float32)
    q_ref[...] = jnp.dot(vnew, swd_ref[...], preferred_element_type=jnp.float32)


def kernel(V, E, weight, Wc, bc, SW, Sb, gamma_V, beta_V, gamma_E, beta_E, edge_index):
    src = edge_index[0]
    dst = edge_index[1]

    ones = jnp.ones((EDGES, 1), jnp.float32)
    cnt = jax.ops.segment_sum(ones, dst, num_segments=N)
    inv_cnt = 1.0 / jnp.maximum(cnt, 1.0)
    has = (cnt > 0)

    def agg(h):
        m = h[src]
        s = jax.ops.segment_sum(m, dst, num_segments=N)
        mean = s * inv_cnt
        mx = jax.ops.segment_max(m, dst, num_segments=N)
        mx = jnp.where(has, mx, 0.0)
        return s, mean, mx

    link_dict = {}
    for s_, d_, w_ in CELL_ARCH:
        link_dict.setdefault(d_, []).append((s_, w_))

    states = [V]
    aggs = {}
    for d_ in range(1, 5):
        acc = jnp.zeros((N, D_V), jnp.float32)
        for s_, w_ in link_dict[d_]:
            if s_ not in aggs:
                aggs[s_] = agg(states[s_])
            ssum, smean, smax = aggs[s_]
            w = weight[w_]
            acc = acc + (w[1] * states[s_] + w[2] * smax + w[3] * smean + w[4] * ssum)
        states.append(acc)

    Vcat = jnp.concatenate(states[1:], axis=1)

    SW_src = SW[:D_V]
    SW_E = SW[D_V:D_V + D_E]
    SW_dst = SW[D_V + D_E:]

    grid = (N // ROW_BLK,)
    Vout, P, Q = pl.pallas_call(
        _final_v_kernel,
        grid=grid,
        in_specs=[
            pl.BlockSpec((ROW_BLK, 4 * D_V), lambda i: (i, 0)),
            pl.BlockSpec((4 * D_V, D_V), lambda i: (0, 0)),
            pl.BlockSpec((D_V,), lambda i: (0,)),
            pl.BlockSpec((D_V,), lambda i: (0,)),
            pl.BlockSpec((D_V,), lambda i: (0,)),
            pl.BlockSpec((ROW_BLK, D_V), lambda i: (i, 0)),
            pl.BlockSpec((D_V, D_E), lambda i: (0, 0)),
            pl.BlockSpec((D_V, D_E), lambda i: (0, 0)),
        ],
        out_specs=[
            pl.BlockSpec((ROW_BLK, D_V), lambda i: (i, 0)),
            pl.BlockSpec((ROW_BLK, D_E), lambda i: (i, 0)),
            pl.BlockSpec((ROW_BLK, D_E), lambda i: (i, 0)),
        ],
        out_shape=[
            jax.ShapeDtypeStruct((N, D_V), jnp.float32),
            jax.ShapeDtypeStruct((N, D_E), jnp.float32),
            jax.ShapeDtypeStruct((N, D_E), jnp.float32),
        ],
    )(Vcat, Wc, bc, gamma_V, beta_V, V, SW_src, SW_dst)

    E_act = jnp.where(E >= 0, E, LEAKY * E)
    E_mid = E_act @ SW_E + Sb
    inv = 1.0 / jnp.sqrt(1.0 + EPS)
    E_new = P[src] + E_mid + Q[dst]
    Ebn = E_new * inv * gamma_E + beta_E
    Eout = jnp.where(Ebn >= 0, Ebn, LEAKY * Ebn) + E
    return Vout, Eout


# speedup vs baseline: 1.5953x; 1.5949x over previous
import functools

import jax
import jax.numpy as jnp
import numpy as np
from jax import lax
from jax.experimental import pallas as pl
from jax.experimental.pallas import tpu as pltpu
from jax.experimental.pallas import tpu_sc as plsc

LEAKY = 0.1
EPS = 1e-5
N = 10000
EDGES = 320000
D_V = 128
D_E = 16
CELL_ARCH = [(0, 1, 0), (0, 2, 1), (1, 2, 2), (0, 3, 3), (2, 3, 4), (0, 4, 5), (3, 4, 6)]

NW = 32
EPW = EDGES // NW
LCAP = 12032
OFFW = 64
ACC_ROWS = 314
CNT_ROWS = 40
BSCALE = 0.0032
CHUNK = 64
ECHUNK = 200

def _wid():
    return lax.axis_index("s") * 2 + lax.axis_index("c")


def _bucket_scalar(d):
    return (d.astype(jnp.float32) * jnp.float32(BSCALE)).astype(jnp.int32)


def _bucket_vec(d16):
    return (d16.astype(jnp.float32) * jnp.float32(BSCALE)).astype(jnp.int32)


def _bucket_body(src_hbm, dst_hbm, bsrc_hbm, bdst_hbm, offs_hbm,
                 src_v, dst_v, lsrc_v, ldst_v, offs_v, idx_v, idx_t, oidx_v, lpos_s):
    w = _wid()
    base = pl.multiple_of(w * EPW, 8)
    pltpu.sync_copy(src_hbm.at[pl.ds(base, EPW)], src_v)
    pltpu.sync_copy(dst_hbm.at[pl.ds(base, EPW)], dst_v)

    izero16 = jnp.zeros((16,), jnp.int32)
    ineg16 = jnp.full((16,), -1, jnp.int32)

    for j in range(NW):
        lpos_s[NW + j] = jnp.int32(0)

    def init_local(i, c):
        lsrc_v[pl.ds(i * 16, 16)] = izero16
        ldst_v[pl.ds(i * 16, 16)] = ineg16
        return c
    lax.fori_loop(0, LCAP // 16, init_local, 0)

    def hist_vec(i, c):
        d16 = dst_v[pl.ds(i * 16, 16)]
        b16 = _bucket_vec(d16)
        for j in range(16):
            bj = b16[j]
            lpos_s[NW + bj] = lpos_s[NW + bj] + 1
        return c
    lax.fori_loop(0, EPW // 16, hist_vec, 0)

    acc = jnp.int32(0)
    starts = []
    for b in range(NW):
        lpos_s[b] = acc
        starts.append(acc)
        acc = acc + ((lpos_s[NW + b] + 63) & ~63)

    starts.append(acc)

    iota16 = lax.iota(jnp.int32, 16)
    for t in range(4):
        e_vec = iota16 + (16 * t)
        oidx = ((e_vec >> 1) << 6) + (e_vec & 1) + (2 * w)
        dat = izero16
        for j in range(16):
            e = 16 * t + j
            dat = jnp.where(iota16 == j, starts[e // 2 + (e % 2)], dat)
        oidx_v[pl.ds(t * 16, 16)] = oidx
        offs_v[pl.ds(t * 16, 16)] = dat
    pltpu.sync_copy(offs_v, offs_hbm.at[oidx_v])

    pltpu.sync_copy(lsrc_v, bsrc_hbm.at[pl.ds(pl.multiple_of(w * LCAP, 8), LCAP)])
    pltpu.sync_copy(ldst_v, bdst_hbm.at[pl.ds(pl.multiple_of(w * LCAP, 8), LCAP)])

    gbase = w * LCAP

    def positions16(i):
        d16 = dst_v[pl.ds(i * 16, 16)]
        b16 = _bucket_vec(d16)
        p16 = izero16
        for j in range(16):
            bj = b16[j]
            pj = lpos_s[bj]
            lpos_s[bj] = pj + 1
            p16 = jnp.where(iota16 == j, pj, p16)
        return p16 + gbase

    def place_group(g, c):
        for j8 in range(8):
            idx_v[pl.ds(j8 * 16, 16)] = positions16(g * 8 + j8)
        pltpu.sync_copy(src_v.at[pl.ds(g * 128, 128)], bsrc_hbm.at[idx_v])
        pltpu.sync_copy(dst_v.at[pl.ds(g * 128, 128)], bdst_hbm.at[idx_v])
        return c
    lax.fori_loop(0, EPW // 128, place_group, 0)

    idx_t[...] = positions16(EPW // 16 - 1)
    pltpu.sync_copy(src_v.at[pl.ds(EPW - 16, 16)], bsrc_hbm.at[idx_t])
    pltpu.sync_copy(dst_v.at[pl.ds(EPW - 16, 16)], bdst_hbm.at[idx_t])


def _bucket_kernel_mk(mesh):
    return functools.partial(
        pl.kernel,
        mesh=mesh,
        out_type=[
        jax.ShapeDtypeStruct((NW * LCAP,), jnp.int32),
        jax.ShapeDtypeStruct((NW * LCAP,), jnp.int32),
        jax.ShapeDtypeStruct((NW * OFFW,), jnp.int32),
        ],
        scratch_types=[
            pltpu.VMEM((EPW,), jnp.int32),
            pltpu.VMEM((EPW,), jnp.int32),
            pltpu.VMEM((LCAP,), jnp.int32),
            pltpu.VMEM((LCAP,), jnp.int32),
            pltpu.VMEM((OFFW,), jnp.int32),
            pltpu.VMEM((128,), jnp.int32),
            pltpu.VMEM((16,), jnp.int32),
            pltpu.VMEM((OFFW,), jnp.int32),
            pltpu.SMEM((2 * NW,), jnp.int32),
        ],
    )(_bucket_body)


def _agg_body(h_hbm, bsrc_hbm, bdst_hbm, offs_hbm, sum_hbm, max_hbm, cnt_hbm,
              offs_v, srcb, dstb, rowb, acc_s, acc_m, acc_c, osm):
    v = _wid()
    rv = (625 * v + 1) // 2

    pltpu.sync_copy(offs_hbm.at[pl.ds(pl.multiple_of(v * OFFW, 8), OFFW)], offs_v)
    for t in range(4):
        ovec = offs_v[pl.ds(t * 16, 16)]
        for j in range(16):
            osm[16 * t + j] = ovec[j]

    zero16 = jnp.zeros((16,), jnp.float32)
    ones16 = jnp.ones((16,), jnp.float32)
    ninf16 = jnp.full((16,), -1e30, jnp.float32)

    def init_acc(i, c):
        for k in range(D_V // 16):
            acc_s[i, pl.ds(k * 16, 16)] = zero16
            acc_m[i, pl.ds(k * 16, 16)] = ninf16
        return c
    lax.fori_loop(0, ACC_ROWS, init_acc, 0)

    def init_cnt(i, c):
        for k in range(D_V // 16):
            acc_c[i, pl.ds(k * 16, 16)] = zero16
        return c
    lax.fori_loop(0, CNT_ROWS, init_cnt, 0)

    def per_group(g, c):
        d16 = dstb[pl.ds(g * 16, 16)]
        dl16 = jnp.where(d16 < 0, jnp.int32(313), d16 - rv)
        for j in range(16):
            dlj = dl16[j]
            co = (dlj & 7) * 16
            acc_c[dlj >> 3, pl.ds(co, 16)] = acc_c[dlj >> 3, pl.ds(co, 16)] + ones16
            for k in range(D_V // 16):
                r = rowb[g * 16 + j, pl.ds(k * 16, 16)]
                acc_s[dlj, pl.ds(k * 16, 16)] = acc_s[dlj, pl.ds(k * 16, 16)] + r
                acc_m[dlj, pl.ds(k * 16, 16)] = jnp.maximum(
                    acc_m[dlj, pl.ds(k * 16, 16)], r)
        return c

    def per_chunk(c, base):
        off = pl.multiple_of(base + c * CHUNK, 8)
        pltpu.sync_copy(bsrc_hbm.at[pl.ds(off, CHUNK)], srcb)
        pltpu.sync_copy(bdst_hbm.at[pl.ds(off, CHUNK)], dstb)
        pltpu.sync_copy(h_hbm.at[srcb], rowb)
        lax.fori_loop(0, CHUNK // 16, per_group, 0)
        return base

    def per_worker(w, c):
        s0 = osm[2 * w]
        s1 = osm[2 * w + 1]
        base = w * LCAP + s0
        nch = (s1 - s0) // CHUNK
        lax.fori_loop(0, nch, per_chunk, base)
        return c
    lax.fori_loop(0, NW, per_worker, 0)

    pltpu.sync_copy(acc_s, sum_hbm.at[v])
    pltpu.sync_copy(acc_m, max_hbm.at[v])
    pltpu.sync_copy(acc_c, cnt_hbm.at[v])


def _agg_kernel_mk(mesh):
    return functools.partial(
        pl.kernel,
        mesh=mesh,
        out_type=[
            jax.ShapeDtypeStruct((NW, ACC_ROWS, D_V), jnp.float32),
            jax.ShapeDtypeStruct((NW, ACC_ROWS, D_V), jnp.float32),
            jax.ShapeDtypeStruct((NW, CNT_ROWS, D_V), jnp.float32),
        ],
        scratch_types=[
            pltpu.VMEM((OFFW,), jnp.int32),
            pltpu.VMEM((CHUNK,), jnp.int32),
            pltpu.VMEM((CHUNK,), jnp.int32),
            pltpu.VMEM((CHUNK, D_V), jnp.float32),
            pltpu.VMEM((ACC_ROWS, D_V), jnp.float32),
            pltpu.VMEM((ACC_ROWS, D_V), jnp.float32),
            pltpu.VMEM((CNT_ROWS, D_V), jnp.float32),
            pltpu.SMEM((OFFW,), jnp.int32),
        ],
    )(_agg_body)


def _edge_body(pq_hbm, src_hbm, dst_hbm, r_hbm, srcb, dstb, pqs, pqd, rb):
    w = _wid()

    def per_row(i, c):
        rb[i, :] = pqs[i, pl.ds(0, 16)] + pqd[i, pl.ds(16, 16)]
        return c

    def per_chunk(c, carry):
        off = pl.multiple_of(w * EPW + c * ECHUNK, 8)
        pltpu.sync_copy(src_hbm.at[pl.ds(off, ECHUNK)], srcb)
        pltpu.sync_copy(dst_hbm.at[pl.ds(off, ECHUNK)], dstb)
        pltpu.sync_copy(pq_hbm.at[srcb], pqs)
        pltpu.sync_copy(pq_hbm.at[dstb], pqd)
        lax.fori_loop(0, ECHUNK, per_row, 0)
        pltpu.sync_copy(rb, r_hbm.at[pl.ds(off, ECHUNK)])
        return carry
    lax.fori_loop(0, EPW // ECHUNK, per_chunk, 0)


def _edge_kernel_mk(mesh):
    return functools.partial(
        pl.kernel,
        mesh=mesh,
        out_type=jax.ShapeDtypeStruct((EDGES, D_E), jnp.float32),
        scratch_types=[
            pltpu.VMEM((ECHUNK,), jnp.int32),
            pltpu.VMEM((ECHUNK,), jnp.int32),
            pltpu.VMEM((ECHUNK, D_V), jnp.float32),
            pltpu.VMEM((ECHUNK, D_V), jnp.float32),
            pltpu.VMEM((ECHUNK, D_E), jnp.float32),
        ],
    )(_edge_body)


@functools.lru_cache(maxsize=1)
def _sc_kernels():
    mesh = plsc.VectorSubcoreMesh(core_axis_name="c", subcore_axis_name="s")
    return _bucket_kernel_mk(mesh), _agg_kernel_mk(mesh), _edge_kernel_mk(mesh)


ROW_BLK = 1000


def _mix1_body(w_ref, cnt_ref, h_ref, s_ref, m_ref, out_ref):
    cnt = cnt_ref[...]
    inv = 1.0 / jnp.maximum(cnt, 1.0)
    has = cnt > 0
    s = s_ref[...]
    mx = jnp.where(has, m_ref[...], 0.0)
    out_ref[...] = (w_ref[0, 1] * h_ref[...] + w_ref[0, 2] * mx
                    + w_ref[0, 3] * (s * inv) + w_ref[0, 4] * s)


def _mix2_body(w_ref, cnt_ref, h0_ref, s0_ref, m0_ref, h1_ref, s1_ref, m1_ref, out_ref):
    cnt = cnt_ref[...]
    inv = 1.0 / jnp.maximum(cnt, 1.0)
    has = cnt > 0
    acc = jnp.zeros((ROW_BLK, D_V), jnp.float32)
    for j, (h_ref, s_ref, m_ref) in enumerate(
            [(h0_ref, s0_ref, m0_ref), (h1_ref, s1_ref, m1_ref)]):
        s = s_ref[...]
        mx = jnp.where(has, m_ref[...], 0.0)
        acc = acc + (w_ref[j, 1] * h_ref[...] + w_ref[j, 2] * mx
                     + w_ref[j, 3] * (s * inv) + w_ref[j, 4] * s)
    out_ref[...] = acc


def _vspec():
    return pl.BlockSpec((ROW_BLK, D_V), lambda i: (i, 0))


def _mix(w_rows, cnt_c, triples):
    n_links = len(triples)
    body = _mix1_body if n_links == 1 else _mix2_body
    in_specs = [
        pl.BlockSpec(memory_space=pltpu.SMEM),
        pl.BlockSpec((ROW_BLK, 1), lambda i: (i, 0)),
    ]
    args = [w_rows, cnt_c]
    for (h, s, m) in triples:
        in_specs += [_vspec(), _vspec(), _vspec()]
        args += [h, s, m]
    return pl.pallas_call(
        body,
        grid=(N // ROW_BLK,),
        in_specs=in_specs,
        out_specs=_vspec(),
        out_shape=jax.ShapeDtypeStruct((N, D_V), jnp.float32),
    )(*args)


def _final_v_body(s1_ref, s2_ref, s3_ref, s4_ref, wc_ref, bc_ref, gv_ref, bv_ref,
                  v_ref, swpq_ref, vout_ref, pq_ref):
    vcat = jnp.concatenate(
        [s1_ref[...], s2_ref[...], s3_ref[...], s4_ref[...]], axis=1)
    vnew = jnp.dot(vcat, wc_ref[...], preferred_element_type=jnp.float32) + bc_ref[...]
    inv = 1.0 / jnp.sqrt(1.0 + EPS)
    vbn = vnew * inv * gv_ref[...] + bv_ref[...]
    vact = jnp.where(vbn >= 0, vbn, LEAKY * vbn)
    vout_ref[...] = vact + v_ref[...]
    pq_ref[...] = jnp.dot(vnew, swpq_ref[...], preferred_element_type=jnp.float32)


EBLK = 8000


def _final_e_body(e_ref, r_ref, swe_ref, sb_ref, ge_ref, be_ref, eout_ref):
    e = e_ref[...]
    e_act = jnp.where(e >= 0, e, LEAKY * e)
    e_mid = jnp.dot(e_act, swe_ref[...], preferred_element_type=jnp.float32) + sb_ref[...]
    inv = 1.0 / jnp.sqrt(1.0 + EPS)
    ebn = (r_ref[...] + e_mid) * inv * ge_ref[...] + be_ref[...]
    eout_ref[...] = jnp.where(ebn >= 0, ebn, LEAKY * ebn) + e


_NN = np.arange(N)
_VB = (_NN * NW) // N
_DL = _NN - (625 * _VB + 1) // 2
_ROW_IDX = np.asarray(_VB * ACC_ROWS + _DL, np.int32)
_CNT_IDX = np.asarray(_VB * (CNT_ROWS * D_V) + (_DL >> 3) * D_V + (_DL & 7) * 16,
                      np.int32)


def kernel(V, E, weight, Wc, bc, SW, Sb, gamma_V, beta_V, gamma_E, beta_E, edge_index):
    src = edge_index[0]
    dst = edge_index[1]

    bucket_k, agg_k, edge_k = _sc_kernels()
    bsrc, bdst, offs = bucket_k(src, dst)
    cnt_c = None

    link_dict = {}
    for s_, d_, w_ in CELL_ARCH:
        link_dict.setdefault(d_, []).append((s_, w_))

    states = [V]
    aggs = {}
    for d_ in range(1, 5):
        triples = []
        w_rows = []
        for s_, w_ in link_dict[d_]:
            if s_ not in aggs:
                ssum, smax, scnt = agg_k(states[s_], bsrc, bdst, offs)
                aggs[s_] = (ssum.reshape(NW * ACC_ROWS, D_V)[_ROW_IDX],
                            smax.reshape(NW * ACC_ROWS, D_V)[_ROW_IDX])
                if cnt_c is None:
                    cnt_c = scnt.reshape(-1)[_CNT_IDX].reshape(N, 1)
            ssum, smax = aggs[s_]
            triples.append((states[s_], ssum, smax))
            w_rows.append(weight[w_])
        states.append(_mix(jnp.stack(w_rows), cnt_c, triples))

    SW_src = SW[:D_V]
    SW_E = SW[D_V:D_V + D_E]
    SW_dst = SW[D_V + D_E:]
    SW_pq = jnp.concatenate(
        [SW_src, SW_dst, jnp.zeros((D_V, D_V - 2 * D_E), jnp.float32)], axis=1)

    Vout, PQ = pl.pallas_call(
        _final_v_body,
        grid=(N // ROW_BLK,),
        in_specs=[
            _vspec(), _vspec(), _vspec(), _vspec(),
            pl.BlockSpec((4 * D_V, D_V), lambda i: (0, 0)),
            pl.BlockSpec((D_V,), lambda i: (0,)),
            pl.BlockSpec((D_V,), lambda i: (0,)),
            pl.BlockSpec((D_V,), lambda i: (0,)),
            _vspec(),
            pl.BlockSpec((D_V, D_V), lambda i: (0, 0)),
        ],
        out_specs=[
            _vspec(),
            _vspec(),
        ],
        out_shape=[
            jax.ShapeDtypeStruct((N, D_V), jnp.float32),
            jax.ShapeDtypeStruct((N, D_V), jnp.float32),
        ],
    )(states[1], states[2], states[3], states[4], Wc, bc, gamma_V, beta_V, V,
      SW_pq)

    R = edge_k(PQ, src, dst)

    Eout = pl.pallas_call(
        _final_e_body,
        grid=(EDGES // EBLK,),
        in_specs=[
            pl.BlockSpec((EBLK, D_E), lambda i: (i, 0)),
            pl.BlockSpec((EBLK, D_E), lambda i: (i, 0)),
            pl.BlockSpec((D_E, D_E), lambda i: (0, 0)),
            pl.BlockSpec((D_E,), lambda i: (0,)),
            pl.BlockSpec((D_E,), lambda i: (0,)),
            pl.BlockSpec((D_E,), lambda i: (0,)),
        ],
        out_specs=pl.BlockSpec((EBLK, D_E), lambda i: (i, 0)),
        out_shape=jax.ShapeDtypeStruct((EDGES, D_E), jnp.float32),
    )(E, R, SW_E, Sb, gamma_E, beta_E)

    return Vout, Eout


# speedup vs baseline: 2.1103x; 1.3228x over previous
import functools

import jax
import jax.numpy as jnp
import numpy as np
from jax import lax
from jax.experimental import pallas as pl
from jax.experimental.pallas import tpu as pltpu
from jax.experimental.pallas import tpu_sc as plsc

LEAKY = 0.1
EPS = 1e-5
N = 10000
EDGES = 320000
D_V = 128
D_E = 16
CELL_ARCH = [(0, 1, 0), (0, 2, 1), (1, 2, 2), (0, 3, 3), (2, 3, 4), (0, 4, 5), (3, 4, 6)]

NW = 32
EPW = EDGES // NW
LCAP = 12032
OFFW = 64
ACC_ROWS = 314
CNT_ROWS = 40
BSCALE = 0.0032
CHUNK = 128
ECHUNK = 200

def _wid():
    return lax.axis_index("s") * 2 + lax.axis_index("c")


def _bucket_scalar(d):
    return (d.astype(jnp.float32) * jnp.float32(BSCALE)).astype(jnp.int32)


def _bucket_vec(d16):
    return (d16.astype(jnp.float32) * jnp.float32(BSCALE)).astype(jnp.int32)


def _bucket_body(src_hbm, dst_hbm, bsrc_hbm, bdst_hbm, offs_hbm,
                 src_v, dst_v, lsrc_v, ldst_v, offs_v, idx_v, idx_t, oidx_v, lpos_s):
    w = _wid()
    base = pl.multiple_of(w * EPW, 8)
    pltpu.sync_copy(src_hbm.at[pl.ds(base, EPW)], src_v)
    pltpu.sync_copy(dst_hbm.at[pl.ds(base, EPW)], dst_v)

    izero16 = jnp.zeros((16,), jnp.int32)
    ineg16 = jnp.full((16,), -1, jnp.int32)

    for j in range(NW):
        lpos_s[NW + j] = jnp.int32(0)

    def init_local(i, c):
        lsrc_v[pl.ds(i * 16, 16)] = izero16
        ldst_v[pl.ds(i * 16, 16)] = ineg16
        return c
    lax.fori_loop(0, LCAP // 16, init_local, 0)

    def hist_vec(i, c):
        d16 = dst_v[pl.ds(i * 16, 16)]
        b16 = _bucket_vec(d16)
        for j in range(16):
            bj = b16[j]
            lpos_s[NW + bj] = lpos_s[NW + bj] + 1
        return c
    lax.fori_loop(0, EPW // 16, hist_vec, 0)

    acc = jnp.int32(0)
    starts = []
    for b in range(NW):
        lpos_s[b] = acc
        starts.append(acc)
        acc = acc + ((lpos_s[NW + b] + 63) & ~63)

    starts.append(acc)

    iota16 = lax.iota(jnp.int32, 16)
    for t in range(4):
        e_vec = iota16 + (16 * t)
        oidx = ((e_vec >> 1) << 6) + (e_vec & 1) + (2 * w)
        dat = izero16
        for j in range(16):
            e = 16 * t + j
            dat = jnp.where(iota16 == j, starts[e // 2 + (e % 2)], dat)
        oidx_v[pl.ds(t * 16, 16)] = oidx
        offs_v[pl.ds(t * 16, 16)] = dat
    pltpu.sync_copy(offs_v, offs_hbm.at[oidx_v])

    pltpu.sync_copy(lsrc_v, bsrc_hbm.at[pl.ds(pl.multiple_of(w * LCAP, 8), LCAP)])
    pltpu.sync_copy(ldst_v, bdst_hbm.at[pl.ds(pl.multiple_of(w * LCAP, 8), LCAP)])

    gbase = w * LCAP

    def positions16(i):
        d16 = dst_v[pl.ds(i * 16, 16)]
        b16 = _bucket_vec(d16)
        p16 = izero16
        for j in range(16):
            bj = b16[j]
            pj = lpos_s[bj]
            lpos_s[bj] = pj + 1
            p16 = jnp.where(iota16 == j, pj, p16)
        return p16 + gbase

    def place_group(g, c):
        for j8 in range(8):
            idx_v[pl.ds(j8 * 16, 16)] = positions16(g * 8 + j8)
        pltpu.sync_copy(src_v.at[pl.ds(g * 128, 128)], bsrc_hbm.at[idx_v])
        pltpu.sync_copy(dst_v.at[pl.ds(g * 128, 128)], bdst_hbm.at[idx_v])
        return c
    lax.fori_loop(0, EPW // 128, place_group, 0)

    idx_t[...] = positions16(EPW // 16 - 1)
    pltpu.sync_copy(src_v.at[pl.ds(EPW - 16, 16)], bsrc_hbm.at[idx_t])
    pltpu.sync_copy(dst_v.at[pl.ds(EPW - 16, 16)], bdst_hbm.at[idx_t])


def _bucket_kernel_mk(mesh):
    return functools.partial(
        pl.kernel,
        mesh=mesh,
        out_type=[
        jax.ShapeDtypeStruct((NW * LCAP,), jnp.int32),
        jax.ShapeDtypeStruct((NW * LCAP,), jnp.int32),
        jax.ShapeDtypeStruct((NW * OFFW,), jnp.int32),
        ],
        scratch_types=[
            pltpu.VMEM((EPW,), jnp.int32),
            pltpu.VMEM((EPW,), jnp.int32),
            pltpu.VMEM((LCAP,), jnp.int32),
            pltpu.VMEM((LCAP,), jnp.int32),
            pltpu.VMEM((OFFW,), jnp.int32),
            pltpu.VMEM((128,), jnp.int32),
            pltpu.VMEM((16,), jnp.int32),
            pltpu.VMEM((OFFW,), jnp.int32),
            pltpu.SMEM((2 * NW,), jnp.int32),
        ],
    )(_bucket_body)


def _agg_body(h_hbm, bsrc_hbm, bdst_hbm, offs_hbm, sum_hbm, max_hbm, cnt_hbm,
              offs_v, srcb, dstb, rowb, acc_s, acc_m, acc_c, osm):
    v = _wid()
    rv = (625 * v + 1) // 2

    pltpu.sync_copy(offs_hbm.at[pl.ds(pl.multiple_of(v * OFFW, 8), OFFW)], offs_v)
    for t in range(4):
        ovec = offs_v[pl.ds(t * 16, 16)]
        for j in range(16):
            osm[16 * t + j] = ovec[j]

    zero16 = jnp.zeros((16,), jnp.float32)
    ones16 = jnp.ones((16,), jnp.float32)
    ninf16 = jnp.full((16,), -1e30, jnp.float32)

    def init_acc(i, c):
        acc_s[pl.ds(i * 16, 16)] = zero16
        acc_m[pl.ds(i * 16, 16)] = ninf16
        return c
    lax.fori_loop(0, ACC_ROWS * D_V // 16, init_acc, 0)

    def init_cnt(i, c):
        acc_c[pl.ds(i * 16, 16)] = zero16
        return c
    lax.fori_loop(0, ACC_ROWS, init_cnt, 0)

    def per_group(g, c):
        d16 = dstb[pl.ds(g * 16, 16)]
        dl16 = jnp.where(d16 < 0, jnp.int32(313), d16 - rv)
        for j in range(16):
            dlj = dl16[j]
            dbase = dlj * D_V
            cbase = dlj * 16
            acc_c[pl.ds(cbase, 16)] = acc_c[pl.ds(cbase, 16)] + ones16
            for k in range(D_V // 16):
                r = rowb[g * 16 + j, pl.ds(k * 16, 16)]
                acc_s[pl.ds(dbase + k * 16, 16)] = acc_s[pl.ds(dbase + k * 16, 16)] + r
                acc_m[pl.ds(dbase + k * 16, 16)] = jnp.maximum(
                    acc_m[pl.ds(dbase + k * 16, 16)], r)
        return c

    def per_chunk(c, base):
        off = pl.multiple_of(base + c * CHUNK, 8)
        pltpu.sync_copy(bsrc_hbm.at[pl.ds(off, CHUNK)], srcb)
        pltpu.sync_copy(bdst_hbm.at[pl.ds(off, CHUNK)], dstb)
        pltpu.sync_copy(h_hbm.at[srcb], rowb)
        lax.fori_loop(0, CHUNK // 16, per_group, 0)
        return base

    def per_worker(w, c):
        s0 = osm[2 * w]
        s1 = osm[2 * w + 1]
        base = w * LCAP + s0
        nch = (s1 - s0) // CHUNK
        lax.fori_loop(0, nch, per_chunk, base)
        return c
    lax.fori_loop(0, NW, per_worker, 0)

    asz = ACC_ROWS * D_V
    csz = ACC_ROWS * 16
    pltpu.sync_copy(acc_s, sum_hbm.at[pl.ds(pl.multiple_of(v * asz, 8), asz)])
    pltpu.sync_copy(acc_m, max_hbm.at[pl.ds(pl.multiple_of(v * asz, 8), asz)])
    pltpu.sync_copy(acc_c, cnt_hbm.at[pl.ds(pl.multiple_of(v * csz, 8), csz)])


def _agg_kernel_mk(mesh):
    return functools.partial(
        pl.kernel,
        mesh=mesh,
        out_type=[
            jax.ShapeDtypeStruct((NW * ACC_ROWS * D_V,), jnp.float32),
            jax.ShapeDtypeStruct((NW * ACC_ROWS * D_V,), jnp.float32),
            jax.ShapeDtypeStruct((NW * ACC_ROWS * 16,), jnp.float32),
        ],
        scratch_types=[
            pltpu.VMEM((OFFW,), jnp.int32),
            pltpu.VMEM((CHUNK,), jnp.int32),
            pltpu.VMEM((CHUNK,), jnp.int32),
            pltpu.VMEM((CHUNK, D_V), jnp.float32),
            pltpu.VMEM((ACC_ROWS * D_V,), jnp.float32),
            pltpu.VMEM((ACC_ROWS * D_V,), jnp.float32),
            pltpu.VMEM((ACC_ROWS * 16,), jnp.float32),
            pltpu.SMEM((OFFW,), jnp.int32),
        ],
    )(_agg_body)


def _edge_body(pq_hbm, src_hbm, dst_hbm, r_hbm, srcb, dstb, pqs, pqd, rb):
    w = _wid()

    def per_row(i, c):
        rb[i, :] = pqs[i, pl.ds(0, 16)] + pqd[i, pl.ds(16, 16)]
        return c

    def per_chunk(c, carry):
        off = pl.multiple_of(w * EPW + c * ECHUNK, 8)
        pltpu.sync_copy(src_hbm.at[pl.ds(off, ECHUNK)], srcb)
        pltpu.sync_copy(dst_hbm.at[pl.ds(off, ECHUNK)], dstb)
        pltpu.sync_copy(pq_hbm.at[srcb], pqs)
        pltpu.sync_copy(pq_hbm.at[dstb], pqd)
        lax.fori_loop(0, ECHUNK, per_row, 0)
        pltpu.sync_copy(rb, r_hbm.at[pl.ds(off, ECHUNK)])
        return carry
    lax.fori_loop(0, EPW // ECHUNK, per_chunk, 0)


def _edge_kernel_mk(mesh):
    return functools.partial(
        pl.kernel,
        mesh=mesh,
        out_type=jax.ShapeDtypeStruct((EDGES, D_E), jnp.float32),
        scratch_types=[
            pltpu.VMEM((ECHUNK,), jnp.int32),
            pltpu.VMEM((ECHUNK,), jnp.int32),
            pltpu.VMEM((ECHUNK, D_V), jnp.float32),
            pltpu.VMEM((ECHUNK, D_V), jnp.float32),
            pltpu.VMEM((ECHUNK, D_E), jnp.float32),
        ],
    )(_edge_body)


@functools.lru_cache(maxsize=1)
def _sc_kernels():
    mesh = plsc.VectorSubcoreMesh(core_axis_name="c", subcore_axis_name="s")
    return _bucket_kernel_mk(mesh), _agg_kernel_mk(mesh), _edge_kernel_mk(mesh)


ROW_BLK = 1000


def _mix1_body(w_ref, cnt_ref, h_ref, s_ref, m_ref, out_ref):
    cnt = cnt_ref[...]
    inv = 1.0 / jnp.maximum(cnt, 1.0)
    has = cnt > 0
    s = s_ref[...]
    mx = jnp.where(has, m_ref[...], 0.0)
    out_ref[...] = (w_ref[0, 1] * h_ref[...] + w_ref[0, 2] * mx
                    + w_ref[0, 3] * (s * inv) + w_ref[0, 4] * s)


def _mix2_body(w_ref, cnt_ref, h0_ref, s0_ref, m0_ref, h1_ref, s1_ref, m1_ref, out_ref):
    cnt = cnt_ref[...]
    inv = 1.0 / jnp.maximum(cnt, 1.0)
    has = cnt > 0
    acc = jnp.zeros((ROW_BLK, D_V), jnp.float32)
    for j, (h_ref, s_ref, m_ref) in enumerate(
            [(h0_ref, s0_ref, m0_ref), (h1_ref, s1_ref, m1_ref)]):
        s = s_ref[...]
        mx = jnp.where(has, m_ref[...], 0.0)
        acc = acc + (w_ref[j, 1] * h_ref[...] + w_ref[j, 2] * mx
                     + w_ref[j, 3] * (s * inv) + w_ref[j, 4] * s)
    out_ref[...] = acc


def _vspec():
    return pl.BlockSpec((ROW_BLK, D_V), lambda i: (i, 0))


def _mix(w_rows, cnt_c, triples):
    n_links = len(triples)
    body = _mix1_body if n_links == 1 else _mix2_body
    in_specs = [
        pl.BlockSpec(memory_space=pltpu.SMEM),
        pl.BlockSpec((ROW_BLK, 1), lambda i: (i, 0)),
    ]
    args = [w_rows, cnt_c]
    for (h, s, m) in triples:
        in_specs += [_vspec(), _vspec(), _vspec()]
        args += [h, s, m]
    return pl.pallas_call(
        body,
        grid=(N // ROW_BLK,),
        in_specs=in_specs,
        out_specs=_vspec(),
        out_shape=jax.ShapeDtypeStruct((N, D_V), jnp.float32),
    )(*args)


def _final_v_body(s1_ref, s2_ref, s3_ref, s4_ref, wc_ref, bc_ref, gv_ref, bv_ref,
                  v_ref, swpq_ref, vout_ref, pq_ref):
    vcat = jnp.concatenate(
        [s1_ref[...], s2_ref[...], s3_ref[...], s4_ref[...]], axis=1)
    vnew = jnp.dot(vcat, wc_ref[...], preferred_element_type=jnp.float32) + bc_ref[...]
    inv = 1.0 / jnp.sqrt(1.0 + EPS)
    vbn = vnew * inv * gv_ref[...] + bv_ref[...]
    vact = jnp.where(vbn >= 0, vbn, LEAKY * vbn)
    vout_ref[...] = vact + v_ref[...]
    pq_ref[...] = jnp.dot(vnew, swpq_ref[...], preferred_element_type=jnp.float32)


EBLK = 8000


def _final_e_body(e_ref, r_ref, swe_ref, sb_ref, ge_ref, be_ref, eout_ref):
    e = e_ref[...]
    e_act = jnp.where(e >= 0, e, LEAKY * e)
    e_mid = jnp.dot(e_act, swe_ref[...], preferred_element_type=jnp.float32) + sb_ref[...]
    inv = 1.0 / jnp.sqrt(1.0 + EPS)
    ebn = (r_ref[...] + e_mid) * inv * ge_ref[...] + be_ref[...]
    eout_ref[...] = jnp.where(ebn >= 0, ebn, LEAKY * ebn) + e


_NN = np.arange(N)
_VB = (_NN * NW) // N
_DL = _NN - (625 * _VB + 1) // 2
_ROW_IDX = np.asarray(_VB * ACC_ROWS + _DL, np.int32)
_CNT_IDX = np.asarray(_VB * (ACC_ROWS * 16) + _DL * 16, np.int32)


def kernel(V, E, weight, Wc, bc, SW, Sb, gamma_V, beta_V, gamma_E, beta_E, edge_index):
    src = edge_index[0]
    dst = edge_index[1]

    bucket_k, agg_k, edge_k = _sc_kernels()
    bsrc, bdst, offs = bucket_k(src, dst)
    cnt_c = None

    link_dict = {}
    for s_, d_, w_ in CELL_ARCH:
        link_dict.setdefault(d_, []).append((s_, w_))

    states = [V]
    aggs = {}
    for d_ in range(1, 5):
        triples = []
        w_rows = []
        for s_, w_ in link_dict[d_]:
            if s_ not in aggs:
                ssum, smax, scnt = agg_k(states[s_], bsrc, bdst, offs)
                aggs[s_] = (ssum.reshape(NW * ACC_ROWS, D_V)[_ROW_IDX],
                            smax.reshape(NW * ACC_ROWS, D_V)[_ROW_IDX])
                if cnt_c is None:
                    cnt_c = scnt.reshape(-1)[_CNT_IDX].reshape(N, 1)
            ssum, smax = aggs[s_]
            triples.append((states[s_], ssum, smax))
            w_rows.append(weight[w_])
        states.append(_mix(jnp.stack(w_rows), cnt_c, triples))

    SW_src = SW[:D_V]
    SW_E = SW[D_V:D_V + D_E]
    SW_dst = SW[D_V + D_E:]
    SW_pq = jnp.concatenate(
        [SW_src, SW_dst, jnp.zeros((D_V, D_V - 2 * D_E), jnp.float32)], axis=1)

    Vout, PQ = pl.pallas_call(
        _final_v_body,
        grid=(N // ROW_BLK,),
        in_specs=[
            _vspec(), _vspec(), _vspec(), _vspec(),
            pl.BlockSpec((4 * D_V, D_V), lambda i: (0, 0)),
            pl.BlockSpec((D_V,), lambda i: (0,)),
            pl.BlockSpec((D_V,), lambda i: (0,)),
            pl.BlockSpec((D_V,), lambda i: (0,)),
            _vspec(),
            pl.BlockSpec((D_V, D_V), lambda i: (0, 0)),
        ],
        out_specs=[
            _vspec(),
            _vspec(),
        ],
        out_shape=[
            jax.ShapeDtypeStruct((N, D_V), jnp.float32),
            jax.ShapeDtypeStruct((N, D_V), jnp.float32),
        ],
    )(states[1], states[2], states[3], states[4], Wc, bc, gamma_V, beta_V, V,
      SW_pq)

    R = edge_k(PQ, src, dst)

    Eout = pl.pallas_call(
        _final_e_body,
        grid=(EDGES // EBLK,),
        in_specs=[
            pl.BlockSpec((EBLK, D_E), lambda i: (i, 0)),
            pl.BlockSpec((EBLK, D_E), lambda i: (i, 0)),
            pl.BlockSpec((D_E, D_E), lambda i: (0, 0)),
            pl.BlockSpec((D_E,), lambda i: (0,)),
            pl.BlockSpec((D_E,), lambda i: (0,)),
            pl.BlockSpec((D_E,), lambda i: (0,)),
        ],
        out_specs=pl.BlockSpec((EBLK, D_E), lambda i: (i, 0)),
        out_shape=jax.ShapeDtypeStruct((EDGES, D_E), jnp.float32),
    )(E, R, SW_E, Sb, gamma_E, beta_E)

    return Vout, Eout
